# Initial kernel scaffold; baseline (speedup 1.0000x reference)
import jax
import jax.numpy as jnp
from jax.experimental import pallas as pl


def kernel(x, gate_w, W1, W2):
    raise NotImplementedError("write your pallas kernel here")


# speedup vs baseline: 1.6083x; 1.6083x over previous
import functools

import jax
import jax.numpy as jnp
from jax import lax
from jax.experimental import pallas as pl
from jax.experimental.pallas import tpu as pltpu
from jax.experimental.pallas import tpu_sc as plsc

E = 8
TOPK = 2
N = 2048
D = 1024
DFF = 4096
A = N * TOPK
BLK = 128
NBLK = (A + E * BLK) // BLK
P = NBLK * BLK
DC = 512
KCH = DFF // DC
PC = 512
PCH = P // PC


def _exclusive_ladder_cumsum(x, axis, length):
    sh = 1
    while sh < length:
        if axis == 0:
            pad = jnp.zeros((sh,) + x.shape[1:], x.dtype)
            x = x + jnp.concatenate([pad, x[:-sh]], axis=0)
        else:
            pad = jnp.zeros(x.shape[:1] + (sh,), x.dtype)
            x = x + jnp.concatenate([pad, x[:, :-sh]], axis=1)
        sh *= 2
    return x


def _gate_body(xf_ref, gwt_ref, dest_ref, tok_ref, ws_ref, eog_ref, nblk_ref,
               dest_sc, tokw_sc):
    g = pl.program_id(0)

    @pl.when(g == 0)
    def _():
        xf = xf_ref[...]
        logits = jnp.dot(xf, gwt_ref[...],
                         preferred_element_type=jnp.float32)
        m = jnp.max(logits, axis=1, keepdims=True)
        p = jnp.exp(logits - m)
        probs = p / jnp.sum(p, axis=1, keepdims=True)

        col = lax.broadcasted_iota(jnp.int32, (N, E), 1)
        m1 = jnp.max(probs, axis=1, keepdims=True)
        a1 = jnp.min(jnp.where(probs == m1, col, E), axis=1, keepdims=True)
        probs2 = jnp.where(col == a1, -1.0, probs)
        m2 = jnp.max(probs2, axis=1, keepdims=True)
        a2 = jnp.min(jnp.where(probs2 == m2, col, E), axis=1, keepdims=True)

        ef = jnp.concatenate([a1, a2], axis=0)
        wf = jnp.concatenate([m1, m2], axis=0)
        oh = (ef == lax.broadcasted_iota(jnp.int32, (A, E), 1)
              ).astype(jnp.float32)
        csum = _exclusive_ladder_cumsum(oh, 0, A)
        rank = jnp.sum(oh * csum, axis=1, keepdims=True) - 1.0
        cnt = csum[A - 1:A, :]
        blocks = jnp.floor((cnt + (BLK - 1)) * (1.0 / BLK))
        ic = _exclusive_ladder_cumsum(blocks, 1, E)
        po = BLK * (ic - blocks)
        dest = rank + jnp.sum(oh * po, axis=1, keepdims=True)
        tb = jnp.sum(blocks)
        cole = lax.broadcasted_iota(jnp.int32, (1, E), 1).astype(jnp.float32)
        le = jnp.max(jnp.where(cnt > 0.0, cole, -1.0))

        grow = lax.broadcasted_iota(jnp.int32, (NBLK, 1), 0).astype(jnp.float32)
        eog_raw = jnp.sum((BLK * ic <= BLK * grow).astype(jnp.float32),
                          axis=1, keepdims=True)
        eog = jnp.where(grow < tb, eog_raw, le)
        eog = jnp.clip(eog, 0.0, float(E - 1))

        irow = lax.broadcasted_iota(jnp.int32, (A, 1), 0)
        tokf = (irow & (N - 1)).astype(jnp.float32)

        dest_sc[...] = dest.astype(jnp.int32)
        tokw_sc[...] = jnp.concatenate([tokf, wf], axis=1)
        dest_ref[...] = dest.astype(jnp.int32)
        eog_ref[...] = eog.astype(jnp.int32)
        nblk_ref[...] = tb.astype(jnp.int32).reshape(1, 1)

    pcols = PC * g + lax.broadcasted_iota(jnp.int32, (1, PC), 1)
    mm = (dest_sc[...] == pcols).astype(jnp.float32)
    tw = tokw_sc[...]
    tok_c = jnp.sum(mm * tw[:, 0:1], axis=0)
    ws_c = jnp.sum(mm * tw[:, 1:2], axis=0)
    tok_ref[...] = tok_c.reshape(PC, 1).astype(jnp.int32)
    ws_ref[...] = ws_c.reshape(PC, 1)


def _gate_call(xf, gwt):
    return pl.pallas_call(
        _gate_body,
        grid=(PCH,),
        in_specs=[
            pl.BlockSpec((N, D), lambda g: (0, 0)),
            pl.BlockSpec((D, E), lambda g: (0, 0)),
        ],
        out_specs=[
            pl.BlockSpec((A, 1), lambda g: (0, 0)),
            pl.BlockSpec((PC, 1), lambda g: (g, 0)),
            pl.BlockSpec((PC, 1), lambda g: (g, 0)),
            pl.BlockSpec((NBLK, 1), lambda g: (0, 0)),
            pl.BlockSpec((1, 1), lambda g: (0, 0)),
        ],
        out_shape=[
            jax.ShapeDtypeStruct((A, 1), jnp.int32),
            jax.ShapeDtypeStruct((P, 1), jnp.int32),
            jax.ShapeDtypeStruct((P, 1), jnp.float32),
            jax.ShapeDtypeStruct((NBLK, 1), jnp.int32),
            jax.ShapeDtypeStruct((1, 1), jnp.int32),
        ],
        scratch_shapes=[
            pltpu.VMEM((A, 1), jnp.int32),
            pltpu.VMEM((A, 2), jnp.float32),
        ],
    )(xf, gwt)


def _ffn_body(eog_ref, nblk_ref, xs_ref, w1_ref, w2_ref, ws_ref, out_ref,
              acc_ref):
    k = pl.program_id(0)
    g = pl.program_id(1)

    @pl.when(g < nblk_ref[0])
    def _():
        xb = xs_ref[...]
        h = lax.dot_general(xb, w1_ref[0], (((1,), (1,)), ((), ())),
                            preferred_element_type=jnp.float32)
        h = 0.5 * h * (1.0 + lax.erf(h * 0.7071067811865476))
        o = lax.dot_general(h, w2_ref[0], (((1,), (1,)), ((), ())),
                            preferred_element_type=jnp.float32)
        base = g * BLK

        @pl.when(k == 0)
        def _():
            acc_ref[pl.ds(base, BLK), :] = o

        @pl.when(k > 0)
        def _():
            acc_ref[pl.ds(base, BLK), :] = acc_ref[pl.ds(base, BLK), :] + o

        @pl.when(k == KCH - 1)
        def _():
            wsv = ws_ref[0, 0, :].reshape(BLK, 1)
            out_ref[...] = acc_ref[pl.ds(base, BLK), :] * wsv


def _ffn_call(eog, nblk, xs, W1, W2, ws3):
    grid_spec = pltpu.PrefetchScalarGridSpec(
        num_scalar_prefetch=2,
        grid=(KCH, NBLK),
        in_specs=[
            pl.BlockSpec((BLK, D), lambda k, g, eog, nblk: (g, 0)),
            pl.BlockSpec((1, DC, D), lambda k, g, eog, nblk: (eog[g], k, 0)),
            pl.BlockSpec((1, D, DC), lambda k, g, eog, nblk: (eog[g], 0, k)),
            pl.BlockSpec((1, 1, BLK), lambda k, g, eog, nblk: (g, 0, 0)),
        ],
        out_specs=pl.BlockSpec((BLK, D), lambda k, g, eog, nblk: (g, 0)),
        scratch_shapes=[pltpu.VMEM((P, D), jnp.float32)],
    )
    return pl.pallas_call(
        _ffn_body,
        grid_spec=grid_spec,
        out_shape=jax.ShapeDtypeStruct((P, D), jnp.float32),
        compiler_params=pltpu.CompilerParams(
            dimension_semantics=("arbitrary", "arbitrary")),
    )(eog, nblk, xs, W1, W2, ws3)


_SC_NC = 2
_SC_NS = 16
_SC_NW = _SC_NC * _SC_NS
_SC_CHUNK = 32


def _sc_gather_rows(table, idx1d, rows, ncols):
    mesh = plsc.VectorSubcoreMesh(core_axis_name="core",
                                  subcore_axis_name="subcore")
    per_w = rows // _SC_NW
    nchunk = per_w // _SC_CHUNK

    @functools.partial(
        pl.kernel,
        out_type=jax.ShapeDtypeStruct((rows, ncols), jnp.float32),
        mesh=mesh,
        scratch_types=[
            pltpu.VMEM((per_w,), jnp.int32),
            pltpu.VMEM((_SC_CHUNK, ncols), jnp.float32),
            pltpu.SemaphoreType.DMA,
        ])
    def kern(x_hbm, i_hbm, o_hbm, idx_v, rows_v, sem):
        wid = lax.axis_index("subcore") * _SC_NC + lax.axis_index("core")
        base = wid * per_w
        pltpu.sync_copy(i_hbm.at[pl.ds(base, per_w)], idx_v)
        for c in range(nchunk):
            pltpu.async_copy(
                x_hbm.at[idx_v.at[pl.ds(c * _SC_CHUNK, _SC_CHUNK)]],
                rows_v, sem).wait()
            pltpu.sync_copy(rows_v, o_hbm.at[pl.ds(base + c * _SC_CHUNK,
                                                   _SC_CHUNK)])

    return kern(table, idx1d)


def _add_body(a_ref, b_ref, o_ref):
    o_ref[...] = a_ref[...] + b_ref[...]


def _add_call(a, b):
    blk = 512
    return pl.pallas_call(
        _add_body,
        grid=(N // blk,),
        in_specs=[
            pl.BlockSpec((blk, D), lambda g: (g, 0)),
            pl.BlockSpec((blk, D), lambda g: (g, 0)),
        ],
        out_specs=pl.BlockSpec((blk, D), lambda g: (g, 0)),
        out_shape=jax.ShapeDtypeStruct((N, D), jnp.float32),
    )(a, b)


def kernel(x, gate_w, W1, W2):
    b, t, h, w, d = x.shape
    xf = x.reshape(N, D)
    gwt = gate_w.T

    dest, tok, ws, eog, nblk = _gate_call(xf, gwt)
    dest = dest.reshape(A)
    eog = eog.reshape(NBLK)
    nblk = nblk.reshape(1)

    xs = _sc_gather_rows(xf, tok.reshape(P), P, D)
    ws3 = ws.reshape(NBLK, 1, BLK)
    out_s = _ffn_call(eog, nblk, xs, W1, W2, ws3)

    r0 = _sc_gather_rows(out_s, dest[:N], N, D)
    r1 = _sc_gather_rows(out_s, dest[N:], N, D)
    y = _add_call(r0, r1)
    return y.reshape(b, t, h, w, d)


# speedup vs baseline: 1.6753x; 1.0417x over previous
import functools

import jax
import jax.numpy as jnp
from jax import lax
from jax.experimental import pallas as pl
from jax.experimental.pallas import tpu as pltpu
from jax.experimental.pallas import tpu_sc as plsc

E = 8
TOPK = 2
N = 2048
D = 1024
DFF = 4096
A = N * TOPK
BLK = 128
NBLK = (A + E * BLK) // BLK
P = NBLK * BLK
DC = 512
KCH = DFF // DC
PC = 512
PCH = P // PC


def _exclusive_ladder_cumsum(x, axis, length):
    sh = 1
    while sh < length:
        if axis == 0:
            pad = jnp.zeros((sh,) + x.shape[1:], x.dtype)
            x = x + jnp.concatenate([pad, x[:-sh]], axis=0)
        else:
            pad = jnp.zeros(x.shape[:1] + (sh,), x.dtype)
            x = x + jnp.concatenate([pad, x[:, :-sh]], axis=1)
        sh *= 2
    return x


def _gate_body(xf_ref, gwt_ref, dest_ref, tok_ref, ws_ref, eog_ref, nblk_ref,
               dest_sc, tokw_sc):
    g = pl.program_id(0)

    @pl.when(g == 0)
    def _():
        xf = xf_ref[...]
        logits = jnp.dot(xf, gwt_ref[...],
                         preferred_element_type=jnp.float32)
        m = jnp.max(logits, axis=1, keepdims=True)
        p = jnp.exp(logits - m)
        probs = p / jnp.sum(p, axis=1, keepdims=True)

        col = lax.broadcasted_iota(jnp.int32, (N, E), 1)
        m1 = jnp.max(probs, axis=1, keepdims=True)
        a1 = jnp.min(jnp.where(probs == m1, col, E), axis=1, keepdims=True)
        probs2 = jnp.where(col == a1, -1.0, probs)
        m2 = jnp.max(probs2, axis=1, keepdims=True)
        a2 = jnp.min(jnp.where(probs2 == m2, col, E), axis=1, keepdims=True)

        ef = jnp.concatenate([a1, a2], axis=0)
        wf = jnp.concatenate([m1, m2], axis=0)
        oh = (ef == lax.broadcasted_iota(jnp.int32, (A, E), 1)
              ).astype(jnp.float32)
        csum = _exclusive_ladder_cumsum(oh, 0, A)
        rank = jnp.sum(oh * csum, axis=1, keepdims=True) - 1.0
        cnt = csum[A - 1:A, :]
        blocks = jnp.floor((cnt + (BLK - 1)) * (1.0 / BLK))
        ic = _exclusive_ladder_cumsum(blocks, 1, E)
        po = BLK * (ic - blocks)
        dest = rank + jnp.sum(oh * po, axis=1, keepdims=True)
        tb = jnp.sum(blocks)
        cole = lax.broadcasted_iota(jnp.int32, (1, E), 1).astype(jnp.float32)
        le = jnp.max(jnp.where(cnt > 0.0, cole, -1.0))

        grow = lax.broadcasted_iota(jnp.int32, (NBLK, 1), 0).astype(jnp.float32)
        eog_raw = jnp.sum((BLK * ic <= BLK * grow).astype(jnp.float32),
                          axis=1, keepdims=True)
        eog = jnp.where(grow < tb, eog_raw, le)
        eog = jnp.clip(eog, 0.0, float(E - 1))

        irow = lax.broadcasted_iota(jnp.int32, (A, 1), 0)
        tokf = (irow & (N - 1)).astype(jnp.float32)

        dest_sc[...] = dest.astype(jnp.int32)
        tokw_sc[...] = jnp.concatenate([tokf, wf], axis=1)
        dest_ref[...] = dest.astype(jnp.int32)
        eog_ref[...] = eog.astype(jnp.int32)
        nblk_ref[...] = tb.astype(jnp.int32).reshape(1, 1)

    pcols = PC * g + lax.broadcasted_iota(jnp.int32, (1, PC), 1)
    mm = (dest_sc[...] == pcols).astype(jnp.float32)
    tw = tokw_sc[...]
    res = lax.dot_general(mm, tw, (((0,), (0,)), ((), ())),
                          preferred_element_type=jnp.float32)
    tok_ref[...] = res[:, 0:1].astype(jnp.int32)
    ws_ref[...] = res[:, 1:2]


def _gate_call(xf, gwt):
    return pl.pallas_call(
        _gate_body,
        grid=(PCH,),
        in_specs=[
            pl.BlockSpec((N, D), lambda g: (0, 0)),
            pl.BlockSpec((D, E), lambda g: (0, 0)),
        ],
        out_specs=[
            pl.BlockSpec((A, 1), lambda g: (0, 0)),
            pl.BlockSpec((PC, 1), lambda g: (g, 0)),
            pl.BlockSpec((PC, 1), lambda g: (g, 0)),
            pl.BlockSpec((NBLK, 1), lambda g: (0, 0)),
            pl.BlockSpec((1, 1), lambda g: (0, 0)),
        ],
        out_shape=[
            jax.ShapeDtypeStruct((A, 1), jnp.int32),
            jax.ShapeDtypeStruct((P, 1), jnp.int32),
            jax.ShapeDtypeStruct((P, 1), jnp.float32),
            jax.ShapeDtypeStruct((NBLK, 1), jnp.int32),
            jax.ShapeDtypeStruct((1, 1), jnp.int32),
        ],
        scratch_shapes=[
            pltpu.VMEM((A, 1), jnp.int32),
            pltpu.VMEM((A, 2), jnp.float32),
        ],
    )(xf, gwt)


def _ffn_body(eog_ref, nblk_ref, xs_ref, w1_ref, w2_ref, ws_ref, out_ref,
              acc_ref, xsb_ref, w1b_ref, w2b_ref):
    k = pl.program_id(0)
    g = pl.program_id(1)
    base = g * BLK

    gprev = jnp.maximum(g - 1, 0)
    wchanged = jnp.logical_or(g == 0, eog_ref[g] != eog_ref[gprev])

    @pl.when(wchanged)
    def _():
        w1b_ref[...] = w1_ref[0].astype(jnp.bfloat16)
        w2b_ref[...] = w2_ref[0].astype(jnp.bfloat16)

    @pl.when(jnp.logical_and(k == 0, g < nblk_ref[0]))
    def _():
        xsb_ref[pl.ds(base, BLK), :] = xs_ref[...].astype(jnp.bfloat16)

    @pl.when(g < nblk_ref[0])
    def _():
        xb = xsb_ref[pl.ds(base, BLK), :]
        h = lax.dot_general(xb, w1b_ref[...], (((1,), (1,)), ((), ())),
                            preferred_element_type=jnp.float32)
        h = 0.5 * h * (1.0 + lax.erf(h * 0.7071067811865476))
        o = lax.dot_general(h.astype(jnp.bfloat16), w2b_ref[...],
                            (((1,), (1,)), ((), ())),
                            preferred_element_type=jnp.float32)

        @pl.when(k == 0)
        def _():
            acc_ref[pl.ds(base, BLK), :] = o

        @pl.when(jnp.logical_and(k > 0, k < KCH - 1))
        def _():
            acc_ref[pl.ds(base, BLK), :] = acc_ref[pl.ds(base, BLK), :] + o

        @pl.when(k == KCH - 1)
        def _():
            wsv = ws_ref[0, 0, :].reshape(BLK, 1)
            out_ref[...] = (acc_ref[pl.ds(base, BLK), :] + o) * wsv


def _ffn_call(eog, nblk, xs, W1, W2, ws3):
    grid_spec = pltpu.PrefetchScalarGridSpec(
        num_scalar_prefetch=2,
        grid=(KCH, NBLK),
        in_specs=[
            pl.BlockSpec((BLK, D),
                         lambda k, g, eog, nblk: (jnp.where(k == 0, g, NBLK - 1), 0)),
            pl.BlockSpec((1, DC, D), lambda k, g, eog, nblk: (eog[g], k, 0)),
            pl.BlockSpec((1, D, DC), lambda k, g, eog, nblk: (eog[g], 0, k)),
            pl.BlockSpec((1, 1, BLK), lambda k, g, eog, nblk: (g, 0, 0)),
        ],
        out_specs=pl.BlockSpec(
            (BLK, D), lambda k, g, eog, nblk: (jnp.where(k == KCH - 1, g, 0), 0)),
        scratch_shapes=[
            pltpu.VMEM((P, D), jnp.float32),
            pltpu.VMEM((P, D), jnp.bfloat16),
            pltpu.VMEM((DC, D), jnp.bfloat16),
            pltpu.VMEM((D, DC), jnp.bfloat16),
        ],
    )
    return pl.pallas_call(
        _ffn_body,
        grid_spec=grid_spec,
        out_shape=jax.ShapeDtypeStruct((P, D), jnp.float32),
        compiler_params=pltpu.CompilerParams(
            dimension_semantics=("arbitrary", "arbitrary")),
    )(eog, nblk, xs, W1, W2, ws3)


_SC_NC = 2
_SC_NS = 16
_SC_NW = _SC_NC * _SC_NS


def _sc_gather_rows(table, idx1d, rows, ncols):
    mesh = plsc.VectorSubcoreMesh(core_axis_name="core",
                                  subcore_axis_name="subcore")
    per_w = rows // _SC_NW
    chunk = 40 if per_w % 40 == 0 else 32
    nchunk = per_w // chunk

    @functools.partial(
        pl.kernel,
        out_type=jax.ShapeDtypeStruct((rows, ncols), jnp.float32),
        mesh=mesh,
        scratch_types=[
            pltpu.VMEM((per_w,), jnp.int32),
            pltpu.VMEM((chunk, ncols), jnp.float32),
            pltpu.VMEM((chunk, ncols), jnp.float32),
            pltpu.SemaphoreType.DMA,
            pltpu.SemaphoreType.DMA,
        ])
    def kern(x_hbm, i_hbm, o_hbm, idx_v, rv0, rv1, sem0, sem1):
        wid = lax.axis_index("subcore") * _SC_NC + lax.axis_index("core")
        base = wid * per_w
        pltpu.sync_copy(i_hbm.at[pl.ds(base, per_w)], idx_v)
        bufs = (rv0, rv1)
        sems = (sem0, sem1)

        def start(c):
            b = c % 2
            return pltpu.async_copy(
                x_hbm.at[idx_v.at[pl.ds(c * chunk, chunk)]], bufs[b], sems[b])

        handles = [start(0)]
        for c in range(nchunk):
            if c + 1 < nchunk:
                handles.append(start(c + 1))
            handles[c].wait()
            pltpu.sync_copy(bufs[c % 2],
                            o_hbm.at[pl.ds(base + c * chunk, chunk)])

    return kern(table, idx1d)


def _add_body(a_ref, b_ref, o_ref):
    o_ref[...] = a_ref[...] + b_ref[...]


def _add_call(a, b):
    blk = 512
    return pl.pallas_call(
        _add_body,
        grid=(N // blk,),
        in_specs=[
            pl.BlockSpec((blk, D), lambda g: (g, 0)),
            pl.BlockSpec((blk, D), lambda g: (g, 0)),
        ],
        out_specs=pl.BlockSpec((blk, D), lambda g: (g, 0)),
        out_shape=jax.ShapeDtypeStruct((N, D), jnp.float32),
    )(a, b)


def kernel(x, gate_w, W1, W2):
    b, t, h, w, d = x.shape
    xf = x.reshape(N, D)
    gwt = gate_w.T

    dest, tok, ws, eog, nblk = _gate_call(xf, gwt)
    dest = dest.reshape(A)
    eog = eog.reshape(NBLK)
    nblk = nblk.reshape(1)

    xs = _sc_gather_rows(xf, tok.reshape(P), P, D)
    ws3 = ws.reshape(NBLK, 1, BLK)
    out_s = _ffn_call(eog, nblk, xs, W1, W2, ws3)

    r0 = _sc_gather_rows(out_s, dest[:N], N, D)
    r1 = _sc_gather_rows(out_s, dest[N:], N, D)
    y = _add_call(r0, r1)
    return y.reshape(b, t, h, w, d)


# speedup vs baseline: 1.9770x; 1.1800x over previous
import functools

import jax
import jax.numpy as jnp
from jax import lax
from jax.experimental import pallas as pl
from jax.experimental.pallas import tpu as pltpu
from jax.experimental.pallas import tpu_sc as plsc

E = 8
TOPK = 2
N = 2048
D = 1024
DFF = 4096
A = N * TOPK
BLK = 128
NBLK = (A + E * BLK) // BLK
P = NBLK * BLK
DC = 1024
KCH = DFF // DC
PC = 512
PCH = P // PC


def _exclusive_ladder_cumsum(x, axis, length):
    sh = 1
    while sh < length:
        if axis == 0:
            pad = jnp.zeros((sh,) + x.shape[1:], x.dtype)
            x = x + jnp.concatenate([pad, x[:-sh]], axis=0)
        else:
            pad = jnp.zeros(x.shape[:1] + (sh,), x.dtype)
            x = x + jnp.concatenate([pad, x[:, :-sh]], axis=1)
        sh *= 2
    return x


def _gate_body(xf_ref, gwt_ref, dest_ref, tok_ref, ws_ref, eog_ref, nblk_ref,
               dest_sc, tokw_sc):
    g = pl.program_id(0)

    @pl.when(g == 0)
    def _():
        xf = xf_ref[...]
        logits = jnp.dot(xf, gwt_ref[...],
                         preferred_element_type=jnp.float32)
        m = jnp.max(logits, axis=1, keepdims=True)
        p = jnp.exp(logits - m)
        probs = p / jnp.sum(p, axis=1, keepdims=True)

        col = lax.broadcasted_iota(jnp.int32, (N, E), 1)
        m1 = jnp.max(probs, axis=1, keepdims=True)
        a1 = jnp.min(jnp.where(probs == m1, col, E), axis=1, keepdims=True)
        probs2 = jnp.where(col == a1, -1.0, probs)
        m2 = jnp.max(probs2, axis=1, keepdims=True)
        a2 = jnp.min(jnp.where(probs2 == m2, col, E), axis=1, keepdims=True)

        ef = jnp.concatenate([a1, a2], axis=0)
        wf = jnp.concatenate([m1, m2], axis=0)
        oh = (ef == lax.broadcasted_iota(jnp.int32, (A, E), 1)
              ).astype(jnp.float32)
        csum = _exclusive_ladder_cumsum(oh, 0, A)
        rank = jnp.sum(oh * csum, axis=1, keepdims=True) - 1.0
        cnt = csum[A - 1:A, :]
        blocks = jnp.floor((cnt + (BLK - 1)) * (1.0 / BLK))
        ic = _exclusive_ladder_cumsum(blocks, 1, E)
        po = BLK * (ic - blocks)
        dest = rank + jnp.sum(oh * po, axis=1, keepdims=True)
        tb = jnp.sum(blocks)
        cole = lax.broadcasted_iota(jnp.int32, (1, E), 1).astype(jnp.float32)
        le = jnp.max(jnp.where(cnt > 0.0, cole, -1.0))

        grow = lax.broadcasted_iota(jnp.int32, (NBLK, 1), 0).astype(jnp.float32)
        eog_raw = jnp.sum((BLK * ic <= BLK * grow).astype(jnp.float32),
                          axis=1, keepdims=True)
        eog = jnp.where(grow < tb, eog_raw, le)
        eog = jnp.clip(eog, 0.0, float(E - 1))

        irow = lax.broadcasted_iota(jnp.int32, (A, 1), 0)
        tokf = (irow & (N - 1)).astype(jnp.float32)

        dest_sc[...] = dest.astype(jnp.int32)
        tokw_sc[...] = jnp.concatenate([tokf, wf], axis=1)
        dest_ref[...] = dest.astype(jnp.int32)
        eog_ref[...] = eog.astype(jnp.int32)
        nblk_ref[...] = tb.astype(jnp.int32).reshape(1, 1)

    pcols = PC * g + lax.broadcasted_iota(jnp.int32, (1, PC), 1)
    mm = (dest_sc[...] == pcols).astype(jnp.float32)
    tw = tokw_sc[...]
    tok_c = jnp.sum(mm * tw[:, 0:1], axis=0)
    ws_c = jnp.sum(mm * tw[:, 1:2], axis=0)
    tok_ref[...] = tok_c.reshape(PC, 1).astype(jnp.int32)
    ws_ref[...] = ws_c.reshape(PC, 1)


def _gate_call(xf, gwt):
    return pl.pallas_call(
        _gate_body,
        grid=(PCH,),
        in_specs=[
            pl.BlockSpec((N, D), lambda g: (0, 0)),
            pl.BlockSpec((D, E), lambda g: (0, 0)),
        ],
        out_specs=[
            pl.BlockSpec((A, 1), lambda g: (0, 0)),
            pl.BlockSpec((PC, 1), lambda g: (g, 0)),
            pl.BlockSpec((PC, 1), lambda g: (g, 0)),
            pl.BlockSpec((NBLK, 1), lambda g: (0, 0)),
            pl.BlockSpec((1, 1), lambda g: (0, 0)),
        ],
        out_shape=[
            jax.ShapeDtypeStruct((A, 1), jnp.int32),
            jax.ShapeDtypeStruct((P, 1), jnp.int32),
            jax.ShapeDtypeStruct((P, 1), jnp.float32),
            jax.ShapeDtypeStruct((NBLK, 1), jnp.int32),
            jax.ShapeDtypeStruct((1, 1), jnp.int32),
        ],
        scratch_shapes=[
            pltpu.VMEM((A, 1), jnp.int32),
            pltpu.VMEM((A, 2), jnp.float32),
        ],
    )(xf, gwt)


def _ffn_body(eog_ref, nblk_ref, xs_ref, w1_ref, w2_ref, ws_ref, out_ref,
              acc_ref, xsb_ref, w1b_ref, w2b_ref):
    k = pl.program_id(0)
    g = pl.program_id(1)
    base = g * BLK

    gprev = jnp.maximum(g - 1, 0)
    wchanged = jnp.logical_or(g == 0, eog_ref[g] != eog_ref[gprev])

    @pl.when(wchanged)
    def _():
        w1b_ref[...] = w1_ref[0].astype(jnp.bfloat16)
        w2b_ref[...] = w2_ref[0].astype(jnp.bfloat16)

    @pl.when(jnp.logical_and(k == 0, g < nblk_ref[0]))
    def _():
        xsb_ref[pl.ds(base, BLK), :] = xs_ref[...].astype(jnp.bfloat16)

    @pl.when(g < nblk_ref[0])
    def _():
        xb = xsb_ref[pl.ds(base, BLK), :]
        h = lax.dot_general(xb, w1b_ref[...], (((1,), (1,)), ((), ())),
                            preferred_element_type=jnp.float32)
        h = 0.5 * h * (1.0 + lax.erf(h * 0.7071067811865476))
        o = lax.dot_general(h.astype(jnp.bfloat16), w2b_ref[...],
                            (((1,), (1,)), ((), ())),
                            preferred_element_type=jnp.float32)

        @pl.when(k == 0)
        def _():
            acc_ref[pl.ds(base, BLK), :] = o

        @pl.when(jnp.logical_and(k > 0, k < KCH - 1))
        def _():
            acc_ref[pl.ds(base, BLK), :] = acc_ref[pl.ds(base, BLK), :] + o

        @pl.when(k == KCH - 1)
        def _():
            wsv = ws_ref[0, 0, :].reshape(BLK, 1)
            out_ref[...] = (acc_ref[pl.ds(base, BLK), :] + o) * wsv


def _ffn_call(eog, nblk, xs, W1, W2, ws3):
    grid_spec = pltpu.PrefetchScalarGridSpec(
        num_scalar_prefetch=2,
        grid=(KCH, NBLK),
        in_specs=[
            pl.BlockSpec((BLK, D),
                         lambda k, g, eog, nblk: (jnp.where(k == 0, g, NBLK - 1), 0)),
            pl.BlockSpec((1, DC, D), lambda k, g, eog, nblk: (eog[g], k, 0)),
            pl.BlockSpec((1, D, DC), lambda k, g, eog, nblk: (eog[g], 0, k)),
            pl.BlockSpec((1, 1, BLK), lambda k, g, eog, nblk: (g, 0, 0)),
        ],
        out_specs=pl.BlockSpec(
            (BLK, D), lambda k, g, eog, nblk: (jnp.where(k == KCH - 1, g, 0), 0)),
        scratch_shapes=[
            pltpu.VMEM((P, D), jnp.float32),
            pltpu.VMEM((P, D), jnp.bfloat16),
            pltpu.VMEM((DC, D), jnp.bfloat16),
            pltpu.VMEM((D, DC), jnp.bfloat16),
        ],
    )
    return pl.pallas_call(
        _ffn_body,
        grid_spec=grid_spec,
        out_shape=jax.ShapeDtypeStruct((P, D), jnp.float32),
        compiler_params=pltpu.CompilerParams(
            dimension_semantics=("arbitrary", "arbitrary")),
    )(eog, nblk, xs, W1, W2, ws3)


_SC_NC = 2
_SC_NS = 16
_SC_NW = _SC_NC * _SC_NS


def _sc_gather_rows(table, idx1d, rows, ncols):
    mesh = plsc.VectorSubcoreMesh(core_axis_name="core",
                                  subcore_axis_name="subcore")
    per_w = rows // _SC_NW
    chunk = 40 if per_w % 40 == 0 else 32
    nchunk = per_w // chunk

    @functools.partial(
        pl.kernel,
        out_type=jax.ShapeDtypeStruct((rows, ncols), jnp.float32),
        mesh=mesh,
        scratch_types=[
            pltpu.VMEM((per_w,), jnp.int32),
            pltpu.VMEM((chunk, ncols), jnp.float32),
            pltpu.VMEM((chunk, ncols), jnp.float32),
            pltpu.SemaphoreType.DMA,
            pltpu.SemaphoreType.DMA,
        ])
    def kern(x_hbm, i_hbm, o_hbm, idx_v, rv0, rv1, sem0, sem1):
        wid = lax.axis_index("subcore") * _SC_NC + lax.axis_index("core")
        base = wid * per_w
        pltpu.sync_copy(i_hbm.at[pl.ds(base, per_w)], idx_v)
        bufs = (rv0, rv1)
        sems = (sem0, sem1)

        def start(c):
            b = c % 2
            return pltpu.async_copy(
                x_hbm.at[idx_v.at[pl.ds(c * chunk, chunk)]], bufs[b], sems[b])

        handles = [start(0)]
        for c in range(nchunk):
            if c + 1 < nchunk:
                handles.append(start(c + 1))
            handles[c].wait()
            pltpu.sync_copy(bufs[c % 2],
                            o_hbm.at[pl.ds(base + c * chunk, chunk)])

    return kern(table, idx1d)


def _add_body(a_ref, b_ref, o_ref):
    o_ref[...] = a_ref[...] + b_ref[...]


def _add_call(a, b):
    blk = 512
    return pl.pallas_call(
        _add_body,
        grid=(N // blk,),
        in_specs=[
            pl.BlockSpec((blk, D), lambda g: (g, 0)),
            pl.BlockSpec((blk, D), lambda g: (g, 0)),
        ],
        out_specs=pl.BlockSpec((blk, D), lambda g: (g, 0)),
        out_shape=jax.ShapeDtypeStruct((N, D), jnp.float32),
    )(a, b)


def kernel(x, gate_w, W1, W2):
    b, t, h, w, d = x.shape
    xf = x.reshape(N, D)
    gwt = gate_w.T

    dest, tok, ws, eog, nblk = _gate_call(xf, gwt)
    dest = dest.reshape(A)
    eog = eog.reshape(NBLK)
    nblk = nblk.reshape(1)

    xs = _sc_gather_rows(xf, tok.reshape(P), P, D)
    ws3 = ws.reshape(NBLK, 1, BLK)
    out_s = _ffn_call(eog, nblk, xs, W1, W2, ws3)

    r0 = _sc_gather_rows(out_s, dest[:N], N, D)
    r1 = _sc_gather_rows(out_s, dest[N:], N, D)
    y = _add_call(r0, r1)
    return y.reshape(b, t, h, w, d)


# speedup vs baseline: 2.2451x; 1.1356x over previous
import functools

import jax
import jax.numpy as jnp
from jax import lax
from jax.experimental import pallas as pl
from jax.experimental.pallas import tpu as pltpu
from jax.experimental.pallas import tpu_sc as plsc

E = 8
TOPK = 2
N = 2048
D = 1024
DFF = 4096
A = N * TOPK
BLK = 128
NBLK = (A + E * BLK) // BLK
P = NBLK * BLK
DC = 1024
KCH = DFF // DC
PC = 512
PCH = P // PC


def _exclusive_ladder_cumsum(x, axis, length):
    sh = 1
    while sh < length:
        if axis == 0:
            pad = jnp.zeros((sh,) + x.shape[1:], x.dtype)
            x = x + jnp.concatenate([pad, x[:-sh]], axis=0)
        else:
            pad = jnp.zeros(x.shape[:1] + (sh,), x.dtype)
            x = x + jnp.concatenate([pad, x[:, :-sh]], axis=1)
        sh *= 2
    return x


def _gate_body(xf_ref, gwt_ref, dest_ref, tok_ref, ws_ref, eog_ref, nblk_ref,
               dest_sc, tokw_sc):
    g = pl.program_id(0)

    @pl.when(g == 0)
    def _():
        xf = xf_ref[...]
        logits = jnp.dot(xf, gwt_ref[...],
                         preferred_element_type=jnp.float32)
        m = jnp.max(logits, axis=1, keepdims=True)
        p = jnp.exp(logits - m)
        probs = p / jnp.sum(p, axis=1, keepdims=True)

        col = lax.broadcasted_iota(jnp.int32, (N, E), 1)
        m1 = jnp.max(probs, axis=1, keepdims=True)
        a1 = jnp.min(jnp.where(probs == m1, col, E), axis=1, keepdims=True)
        probs2 = jnp.where(col == a1, -1.0, probs)
        m2 = jnp.max(probs2, axis=1, keepdims=True)
        a2 = jnp.min(jnp.where(probs2 == m2, col, E), axis=1, keepdims=True)

        ef = jnp.concatenate([a1, a2], axis=0)
        wf = jnp.concatenate([m1, m2], axis=0)
        oh = (ef == lax.broadcasted_iota(jnp.int32, (A, E), 1)
              ).astype(jnp.float32)
        csum = _exclusive_ladder_cumsum(oh, 0, A)
        rank = jnp.sum(oh * csum, axis=1, keepdims=True) - 1.0
        cnt = csum[A - 1:A, :]
        blocks = jnp.floor((cnt + (BLK - 1)) * (1.0 / BLK))
        ic = _exclusive_ladder_cumsum(blocks, 1, E)
        po = BLK * (ic - blocks)
        dest = rank + jnp.sum(oh * po, axis=1, keepdims=True)
        tb = jnp.sum(blocks)
        cole = lax.broadcasted_iota(jnp.int32, (1, E), 1).astype(jnp.float32)
        le = jnp.max(jnp.where(cnt > 0.0, cole, -1.0))

        grow = lax.broadcasted_iota(jnp.int32, (NBLK, 1), 0).astype(jnp.float32)
        eog_raw = jnp.sum((BLK * ic <= BLK * grow).astype(jnp.float32),
                          axis=1, keepdims=True)
        eog = jnp.where(grow < tb, eog_raw, le)
        eog = jnp.clip(eog, 0.0, float(E - 1))

        irow = lax.broadcasted_iota(jnp.int32, (A, 1), 0)
        tokf = (irow & (N - 1)).astype(jnp.float32)

        dest_sc[...] = dest.astype(jnp.int32)
        tokw_sc[...] = jnp.concatenate([tokf, wf], axis=1)
        dest_ref[...] = dest.astype(jnp.int32)
        eog_ref[...] = eog.astype(jnp.int32)
        nblk_ref[...] = tb.astype(jnp.int32).reshape(1, 1)

    pcols = PC * g + lax.broadcasted_iota(jnp.int32, (1, PC), 1)
    mm = (dest_sc[...] == pcols).astype(jnp.float32)
    tw = tokw_sc[...]
    hit = jnp.sum(mm, axis=0)
    tok_c = jnp.sum(mm * tw[:, 0:1], axis=0)
    ws_c = jnp.sum(mm * tw[:, 1:2], axis=0)
    dummy = (pcols[0, :] & (N - 1)).astype(jnp.float32)
    tok_c = tok_c + (1.0 - hit) * dummy
    tok_ref[...] = tok_c.reshape(PC, 1).astype(jnp.int32)
    ws_ref[...] = ws_c.reshape(PC, 1)


def _gate_call(xf, gwt):
    return pl.pallas_call(
        _gate_body,
        grid=(PCH,),
        in_specs=[
            pl.BlockSpec((N, D), lambda g: (0, 0)),
            pl.BlockSpec((D, E), lambda g: (0, 0)),
        ],
        out_specs=[
            pl.BlockSpec((A, 1), lambda g: (0, 0)),
            pl.BlockSpec((PC, 1), lambda g: (g, 0)),
            pl.BlockSpec((PC, 1), lambda g: (g, 0)),
            pl.BlockSpec((NBLK, 1), lambda g: (0, 0)),
            pl.BlockSpec((1, 1), lambda g: (0, 0)),
        ],
        out_shape=[
            jax.ShapeDtypeStruct((A, 1), jnp.int32),
            jax.ShapeDtypeStruct((P, 1), jnp.int32),
            jax.ShapeDtypeStruct((P, 1), jnp.float32),
            jax.ShapeDtypeStruct((NBLK, 1), jnp.int32),
            jax.ShapeDtypeStruct((1, 1), jnp.int32),
        ],
        scratch_shapes=[
            pltpu.VMEM((A, 1), jnp.int32),
            pltpu.VMEM((A, 2), jnp.float32),
        ],
    )(xf, gwt)


def _ffn_body(eog_ref, nblk_ref, xs_ref, w1_ref, w2_ref, ws_ref, out_ref,
              acc_ref, xsb_ref, w1b_ref, w2b_ref):
    k = pl.program_id(0)
    g = pl.program_id(1)
    base = g * BLK

    gprev = jnp.maximum(g - 1, 0)
    wchanged = jnp.logical_or(g == 0, eog_ref[g] != eog_ref[gprev])

    @pl.when(wchanged)
    def _():
        w1b_ref[...] = w1_ref[0].astype(jnp.bfloat16)
        w2b_ref[...] = w2_ref[0].astype(jnp.bfloat16)

    @pl.when(jnp.logical_and(k == 0, g < nblk_ref[0]))
    def _():
        xsb_ref[pl.ds(base, BLK), :] = xs_ref[...].astype(jnp.bfloat16)

    @pl.when(g < nblk_ref[0])
    def _():
        xb = xsb_ref[pl.ds(base, BLK), :]
        h = lax.dot_general(xb, w1b_ref[...], (((1,), (1,)), ((), ())),
                            preferred_element_type=jnp.float32)
        h = 0.5 * h * (1.0 + lax.erf(h * 0.7071067811865476))
        o = lax.dot_general(h.astype(jnp.bfloat16), w2b_ref[...],
                            (((1,), (1,)), ((), ())),
                            preferred_element_type=jnp.float32)

        @pl.when(k == 0)
        def _():
            acc_ref[pl.ds(base, BLK), :] = o

        @pl.when(jnp.logical_and(k > 0, k < KCH - 1))
        def _():
            acc_ref[pl.ds(base, BLK), :] = acc_ref[pl.ds(base, BLK), :] + o

        @pl.when(k == KCH - 1)
        def _():
            wsv = ws_ref[0, 0, :].reshape(BLK, 1)
            out_ref[...] = (acc_ref[pl.ds(base, BLK), :] + o) * wsv


def _ffn_call(eog, nblk, xs, W1, W2, ws3):
    grid_spec = pltpu.PrefetchScalarGridSpec(
        num_scalar_prefetch=2,
        grid=(KCH, NBLK),
        in_specs=[
            pl.BlockSpec((BLK, D),
                         lambda k, g, eog, nblk: (jnp.where(k == 0, g, NBLK - 1), 0)),
            pl.BlockSpec((1, DC, D), lambda k, g, eog, nblk: (eog[g], k, 0)),
            pl.BlockSpec((1, D, DC), lambda k, g, eog, nblk: (eog[g], 0, k)),
            pl.BlockSpec((1, 1, BLK), lambda k, g, eog, nblk: (g, 0, 0)),
        ],
        out_specs=pl.BlockSpec(
            (BLK, D), lambda k, g, eog, nblk: (jnp.where(k == KCH - 1, g, 0), 0)),
        scratch_shapes=[
            pltpu.VMEM((P, D), jnp.float32),
            pltpu.VMEM((P, D), jnp.bfloat16),
            pltpu.VMEM((DC, D), jnp.bfloat16),
            pltpu.VMEM((D, DC), jnp.bfloat16),
        ],
    )
    return pl.pallas_call(
        _ffn_body,
        grid_spec=grid_spec,
        out_shape=jax.ShapeDtypeStruct((P, D), jnp.float32),
        compiler_params=pltpu.CompilerParams(
            dimension_semantics=("arbitrary", "arbitrary")),
    )(eog, nblk, xs, W1, W2, ws3)


_SC_NC = 2
_SC_NS = 16
_SC_NW = _SC_NC * _SC_NS


def _sc_gather_rows(table, idx1d, rows, ncols, dtype):
    mesh = plsc.VectorSubcoreMesh(core_axis_name="core",
                                  subcore_axis_name="subcore")
    per_w = rows // _SC_NW
    bytes_per_row = ncols * jnp.dtype(dtype).itemsize
    max_chunk = (160 * 1024) // bytes_per_row
    chunk = max(c for c in (8, 16, 24, 32, 40, 64, 80)
                if c <= max_chunk and per_w % c == 0)
    nchunk = per_w // chunk

    @functools.partial(
        pl.kernel,
        out_type=jax.ShapeDtypeStruct((rows, ncols), dtype),
        mesh=mesh,
        scratch_types=[
            pltpu.VMEM((per_w,), jnp.int32),
            pltpu.VMEM((chunk, ncols), dtype),
            pltpu.VMEM((chunk, ncols), dtype),
            pltpu.SemaphoreType.DMA,
            pltpu.SemaphoreType.DMA,
        ])
    def kern(x_hbm, i_hbm, o_hbm, idx_v, rv0, rv1, sem0, sem1):
        wid = lax.axis_index("subcore") * _SC_NC + lax.axis_index("core")
        base = wid * per_w
        pltpu.sync_copy(i_hbm.at[pl.ds(base, per_w)], idx_v)
        bufs = (rv0, rv1)
        sems = (sem0, sem1)

        def start(c):
            b = c % 2
            return pltpu.async_copy(
                x_hbm.at[idx_v.at[pl.ds(c * chunk, chunk)]], bufs[b], sems[b])

        handles = [start(0)]
        for c in range(nchunk):
            if c + 1 < nchunk:
                handles.append(start(c + 1))
            handles[c].wait()
            pltpu.sync_copy(bufs[c % 2],
                            o_hbm.at[pl.ds(base + c * chunk, chunk)])

    return kern(table, idx1d)


def _add_body(a_ref, b_ref, o_ref):
    o_ref[...] = a_ref[...] + b_ref[...]


_ADD_BLK = 512


def _add_call(r):
    return pl.pallas_call(
        _add_body,
        grid=(N // _ADD_BLK,),
        in_specs=[
            pl.BlockSpec((_ADD_BLK, D), lambda g: (g, 0)),
            pl.BlockSpec((_ADD_BLK, D), lambda g: (g + N // _ADD_BLK, 0)),
        ],
        out_specs=pl.BlockSpec((_ADD_BLK, D), lambda g: (g, 0)),
        out_shape=jax.ShapeDtypeStruct((N, D), jnp.float32),
    )(r, r)


def kernel(x, gate_w, W1, W2):
    b, t, h, w, d = x.shape
    xf = x.reshape(N, D)
    gwt = gate_w.T

    dest, tok, ws, eog, nblk = _gate_call(xf, gwt)
    dest = dest.reshape(A)
    eog = eog.reshape(NBLK)
    nblk = nblk.reshape(1)

    xs = _sc_gather_rows(xf, tok.reshape(P), P, D, jnp.float32)
    ws3 = ws.reshape(NBLK, 1, BLK)
    out_s = _ffn_call(eog, nblk, xs, W1, W2, ws3)

    r = _sc_gather_rows(out_s, dest, A, D, jnp.float32)
    y = _add_call(r)
    return y.reshape(b, t, h, w, d)


# speedup vs baseline: 2.8619x; 1.2747x over previous
import functools

import jax
import jax.numpy as jnp
from jax import lax
from jax.experimental import pallas as pl
from jax.experimental.pallas import tpu as pltpu
from jax.experimental.pallas import tpu_sc as plsc

E = 8
TOPK = 2
N = 2048
D = 1024
DFF = 4096
A = N * TOPK
BLK = 256
NBLK = (A + E * BLK) // BLK
P = NBLK * BLK
DC = 1024
KCH = DFF // DC
PC = 512
PCH = P // PC


def _exclusive_ladder_cumsum(x, axis, length):
    sh = 1
    while sh < length:
        if axis == 0:
            pad = jnp.zeros((sh,) + x.shape[1:], x.dtype)
            x = x + jnp.concatenate([pad, x[:-sh]], axis=0)
        else:
            pad = jnp.zeros(x.shape[:1] + (sh,), x.dtype)
            x = x + jnp.concatenate([pad, x[:, :-sh]], axis=1)
        sh *= 2
    return x


def _gate_body(xf_ref, gwt_ref, dest_ref, tok_ref, ws_ref, eog_ref, nblk_ref,
               dest_sc, tokw_sc):
    g = pl.program_id(0)

    @pl.when(g == 0)
    def _():
        xf = xf_ref[...]
        logits = jnp.dot(xf, gwt_ref[...],
                         preferred_element_type=jnp.float32)
        m = jnp.max(logits, axis=1, keepdims=True)
        p = jnp.exp(logits - m)
        probs = p / jnp.sum(p, axis=1, keepdims=True)

        col = lax.broadcasted_iota(jnp.int32, (N, E), 1)
        m1 = jnp.max(probs, axis=1, keepdims=True)
        a1 = jnp.min(jnp.where(probs == m1, col, E), axis=1, keepdims=True)
        probs2 = jnp.where(col == a1, -1.0, probs)
        m2 = jnp.max(probs2, axis=1, keepdims=True)
        a2 = jnp.min(jnp.where(probs2 == m2, col, E), axis=1, keepdims=True)

        ef = jnp.concatenate([a1, a2], axis=0)
        wf = jnp.concatenate([m1, m2], axis=0)
        oh = (ef == lax.broadcasted_iota(jnp.int32, (A, E), 1)
              ).astype(jnp.float32)
        csum = _exclusive_ladder_cumsum(oh, 0, A)
        rank = jnp.sum(oh * csum, axis=1, keepdims=True) - 1.0
        cnt = csum[A - 1:A, :]
        blocks = jnp.floor((cnt + (BLK - 1)) * (1.0 / BLK))
        ic = _exclusive_ladder_cumsum(blocks, 1, E)
        po = BLK * (ic - blocks)
        dest = rank + jnp.sum(oh * po, axis=1, keepdims=True)
        tb = jnp.sum(blocks)
        cole = lax.broadcasted_iota(jnp.int32, (1, E), 1).astype(jnp.float32)
        le = jnp.max(jnp.where(cnt > 0.0, cole, -1.0))

        grow = lax.broadcasted_iota(jnp.int32, (NBLK, 1), 0).astype(jnp.float32)
        eog_raw = jnp.sum((BLK * ic <= BLK * grow).astype(jnp.float32),
                          axis=1, keepdims=True)
        eog = jnp.where(grow < tb, eog_raw, le)
        eog = jnp.clip(eog, 0.0, float(E - 1))

        irow = lax.broadcasted_iota(jnp.int32, (A, 1), 0)
        tokf = (irow & (N - 1)).astype(jnp.float32)

        dest_sc[...] = dest.astype(jnp.int32)
        tokw_sc[...] = jnp.concatenate([tokf, wf], axis=1)
        dest_ref[...] = dest.astype(jnp.int32)
        eog_ref[...] = eog.astype(jnp.int32)
        nblk_ref[...] = tb.astype(jnp.int32).reshape(1, 1)

    pcols = PC * g + lax.broadcasted_iota(jnp.int32, (1, PC), 1)
    mm = (dest_sc[...] == pcols).astype(jnp.float32)
    tw = tokw_sc[...]
    hit = jnp.sum(mm, axis=0)
    tok_c = jnp.sum(mm * tw[:, 0:1], axis=0)
    ws_c = jnp.sum(mm * tw[:, 1:2], axis=0)
    dummy = (pcols[0, :] & (N - 1)).astype(jnp.float32)
    tok_c = tok_c + (1.0 - hit) * dummy
    tok_ref[...] = tok_c.reshape(PC, 1).astype(jnp.int32)
    ws_ref[...] = ws_c.reshape(PC, 1)


def _gate_call(xf, gwt):
    return pl.pallas_call(
        _gate_body,
        grid=(PCH,),
        in_specs=[
            pl.BlockSpec((N, D), lambda g: (0, 0)),
            pl.BlockSpec((D, E), lambda g: (0, 0)),
        ],
        out_specs=[
            pl.BlockSpec((A, 1), lambda g: (0, 0)),
            pl.BlockSpec((PC, 1), lambda g: (g, 0)),
            pl.BlockSpec((PC, 1), lambda g: (g, 0)),
            pl.BlockSpec((NBLK, 1), lambda g: (0, 0)),
            pl.BlockSpec((1, 1), lambda g: (0, 0)),
        ],
        out_shape=[
            jax.ShapeDtypeStruct((A, 1), jnp.int32),
            jax.ShapeDtypeStruct((P, 1), jnp.int32),
            jax.ShapeDtypeStruct((P, 1), jnp.float32),
            jax.ShapeDtypeStruct((NBLK, 1), jnp.int32),
            jax.ShapeDtypeStruct((1, 1), jnp.int32),
        ],
        scratch_shapes=[
            pltpu.VMEM((A, 1), jnp.int32),
            pltpu.VMEM((A, 2), jnp.float32),
        ],
    )(xf, gwt)


def _ffn_body(eog_ref, nblk_ref, xs_ref, w1_ref, w2_ref, ws_ref, out_ref,
              acc_ref, w1b_ref, w2b_ref):
    k = pl.program_id(0)
    g = pl.program_id(1)
    base = g * BLK

    gprev = jnp.maximum(g - 1, 0)
    wchanged = jnp.logical_or(g == 0, eog_ref[g] != eog_ref[gprev])

    @pl.when(wchanged)
    def _():
        w1b_ref[...] = w1_ref[0].astype(jnp.bfloat16)
        w2b_ref[...] = w2_ref[0].astype(jnp.bfloat16)

    @pl.when(g < nblk_ref[0])
    def _():
        xb = xs_ref[...].astype(jnp.bfloat16)
        h = lax.dot_general(xb, w1b_ref[...], (((1,), (1,)), ((), ())),
                            preferred_element_type=jnp.float32)
        h = 0.5 * h * (1.0 + lax.erf(h * 0.7071067811865476))
        o = lax.dot_general(h.astype(jnp.bfloat16), w2b_ref[...],
                            (((1,), (1,)), ((), ())),
                            preferred_element_type=jnp.float32)

        @pl.when(k == 0)
        def _():
            acc_ref[pl.ds(base, BLK), :] = o

        @pl.when(jnp.logical_and(k > 0, k < KCH - 1))
        def _():
            acc_ref[pl.ds(base, BLK), :] = acc_ref[pl.ds(base, BLK), :] + o

        @pl.when(k == KCH - 1)
        def _():
            wsv = ws_ref[0, 0, :].reshape(BLK, 1)
            out_ref[...] = (acc_ref[pl.ds(base, BLK), :] + o) * wsv


def _ffn_call(eog, nblk, xs, W1, W2, ws3):
    grid_spec = pltpu.PrefetchScalarGridSpec(
        num_scalar_prefetch=2,
        grid=(KCH, NBLK),
        in_specs=[
            pl.BlockSpec((BLK, D), lambda k, g, eog, nblk: (g, 0)),
            pl.BlockSpec((1, DC, D), lambda k, g, eog, nblk: (eog[g], k, 0)),
            pl.BlockSpec((1, D, DC), lambda k, g, eog, nblk: (eog[g], 0, k)),
            pl.BlockSpec((1, 1, BLK), lambda k, g, eog, nblk: (g, 0, 0)),
        ],
        out_specs=pl.BlockSpec(
            (BLK, D), lambda k, g, eog, nblk: (jnp.where(k == KCH - 1, g, 0), 0)),
        scratch_shapes=[
            pltpu.VMEM((P, D), jnp.float32),
            pltpu.VMEM((DC, D), jnp.bfloat16),
            pltpu.VMEM((D, DC), jnp.bfloat16),
        ],
    )
    return pl.pallas_call(
        _ffn_body,
        grid_spec=grid_spec,
        out_shape=jax.ShapeDtypeStruct((P, D), jnp.float32),
        compiler_params=pltpu.CompilerParams(
            dimension_semantics=("arbitrary", "arbitrary")),
    )(eog, nblk, xs, W1, W2, ws3)


_SC_NC = 2
_SC_NS = 16
_SC_NW = _SC_NC * _SC_NS


def _sc_gather_rows(table, idx1d, rows, ncols, dtype):
    mesh = plsc.VectorSubcoreMesh(core_axis_name="core",
                                  subcore_axis_name="subcore")
    per_w = rows // _SC_NW
    bytes_per_row = ncols * jnp.dtype(dtype).itemsize
    max_chunk = (160 * 1024) // bytes_per_row
    chunk = max(c for c in (8, 16, 24, 32, 40, 64, 80)
                if c <= max_chunk and per_w % c == 0)
    nchunk = per_w // chunk

    @functools.partial(
        pl.kernel,
        out_type=jax.ShapeDtypeStruct((rows, ncols), dtype),
        mesh=mesh,
        scratch_types=[
            pltpu.VMEM((per_w,), jnp.int32),
            pltpu.VMEM((chunk, ncols), dtype),
            pltpu.VMEM((chunk, ncols), dtype),
            pltpu.SemaphoreType.DMA,
            pltpu.SemaphoreType.DMA,
        ])
    def kern(x_hbm, i_hbm, o_hbm, idx_v, rv0, rv1, sem0, sem1):
        wid = lax.axis_index("subcore") * _SC_NC + lax.axis_index("core")
        base = wid * per_w
        pltpu.sync_copy(i_hbm.at[pl.ds(base, per_w)], idx_v)
        bufs = (rv0, rv1)
        sems = (sem0, sem1)

        def start(c):
            b = c % 2
            return pltpu.async_copy(
                x_hbm.at[idx_v.at[pl.ds(c * chunk, chunk)]], bufs[b], sems[b])

        handles = [start(0)]
        for c in range(nchunk):
            if c + 1 < nchunk:
                handles.append(start(c + 1))
            handles[c].wait()
            pltpu.sync_copy(bufs[c % 2],
                            o_hbm.at[pl.ds(base + c * chunk, chunk)])

    return kern(table, idx1d)


def _add_body(a_ref, b_ref, o_ref):
    o_ref[...] = a_ref[...] + b_ref[...]


_ADD_BLK = 512


def _add_call(r):
    return pl.pallas_call(
        _add_body,
        grid=(N // _ADD_BLK,),
        in_specs=[
            pl.BlockSpec((_ADD_BLK, D), lambda g: (g, 0)),
            pl.BlockSpec((_ADD_BLK, D), lambda g: (g + N // _ADD_BLK, 0)),
        ],
        out_specs=pl.BlockSpec((_ADD_BLK, D), lambda g: (g, 0)),
        out_shape=jax.ShapeDtypeStruct((N, D), jnp.float32),
    )(r, r)


def kernel(x, gate_w, W1, W2):
    b, t, h, w, d = x.shape
    xf = x.reshape(N, D)
    gwt = gate_w.T

    dest, tok, ws, eog, nblk = _gate_call(xf, gwt)
    dest = dest.reshape(A)
    eog = eog.reshape(NBLK)
    nblk = nblk.reshape(1)

    xs = _sc_gather_rows(xf, tok.reshape(P), P, D, jnp.float32)
    ws3 = ws.reshape(NBLK, 1, BLK)
    out_s = _ffn_call(eog, nblk, xs, W1, W2, ws3)

    r = _sc_gather_rows(out_s, dest, A, D, jnp.float32)
    y = _add_call(r)
    return y.reshape(b, t, h, w, d)


# speedup vs baseline: 2.9983x; 1.0477x over previous
import functools

import jax
import jax.numpy as jnp
from jax import lax
from jax.experimental import pallas as pl
from jax.experimental.pallas import tpu as pltpu
from jax.experimental.pallas import tpu_sc as plsc

E = 8
TOPK = 2
N = 2048
D = 1024
DFF = 4096
A = N * TOPK
BLK = 256
NBLK = (A + E * BLK) // BLK
P = NBLK * BLK
DC = 1024
KCH = DFF // DC
PC = 512
PCH = P // PC


def _exclusive_ladder_cumsum(x, axis, length):
    sh = 1
    while sh < length:
        if axis == 0:
            pad = jnp.zeros((sh,) + x.shape[1:], x.dtype)
            x = x + jnp.concatenate([pad, x[:-sh]], axis=0)
        else:
            pad = jnp.zeros(x.shape[:1] + (sh,), x.dtype)
            x = x + jnp.concatenate([pad, x[:, :-sh]], axis=1)
        sh *= 2
    return x


def _gate_body(xf_ref, gwt_ref, dest_ref, tok_ref, ws_ref, eog_ref, nblk_ref,
               dest_sc, tokw_sc):
    g = pl.program_id(0)

    @pl.when(g == 0)
    def _():
        xf = xf_ref[...]
        logits = jnp.dot(xf, gwt_ref[...],
                         preferred_element_type=jnp.float32)
        m = jnp.max(logits, axis=1, keepdims=True)
        p = jnp.exp(logits - m)
        probs = p / jnp.sum(p, axis=1, keepdims=True)

        col = lax.broadcasted_iota(jnp.int32, (N, E), 1)
        m1 = jnp.max(probs, axis=1, keepdims=True)
        a1 = jnp.min(jnp.where(probs == m1, col, E), axis=1, keepdims=True)
        probs2 = jnp.where(col == a1, -1.0, probs)
        m2 = jnp.max(probs2, axis=1, keepdims=True)
        a2 = jnp.min(jnp.where(probs2 == m2, col, E), axis=1, keepdims=True)

        ef = jnp.concatenate([a1, a2], axis=0)
        wf = jnp.concatenate([m1, m2], axis=0)
        oh = (ef == lax.broadcasted_iota(jnp.int32, (A, E), 1)
              ).astype(jnp.float32)
        csum = _exclusive_ladder_cumsum(oh, 0, A)
        rank = jnp.sum(oh * csum, axis=1, keepdims=True) - 1.0
        cnt = csum[A - 1:A, :]
        blocks = jnp.floor((cnt + (BLK - 1)) * (1.0 / BLK))
        ic = _exclusive_ladder_cumsum(blocks, 1, E)
        po = BLK * (ic - blocks)
        dest = rank + jnp.sum(oh * po, axis=1, keepdims=True)
        tb = jnp.sum(blocks)
        cole = lax.broadcasted_iota(jnp.int32, (1, E), 1).astype(jnp.float32)
        le = jnp.max(jnp.where(cnt > 0.0, cole, -1.0))

        grow = lax.broadcasted_iota(jnp.int32, (NBLK, 1), 0).astype(jnp.float32)
        eog_raw = jnp.sum((BLK * ic <= BLK * grow).astype(jnp.float32),
                          axis=1, keepdims=True)
        eog = jnp.where(grow < tb, eog_raw, le)
        eog = jnp.clip(eog, 0.0, float(E - 1))

        irow = lax.broadcasted_iota(jnp.int32, (A, 1), 0)
        tokf = (irow & (N - 1)).astype(jnp.float32)

        dest_sc[...] = dest.astype(jnp.int32)
        tokw_sc[...] = jnp.concatenate([tokf, wf], axis=1)
        dest_ref[...] = dest.astype(jnp.int32)
        eog_ref[...] = eog.astype(jnp.int32)
        nblk_ref[...] = tb.astype(jnp.int32).reshape(1, 1)

    pcols = PC * g + lax.broadcasted_iota(jnp.int32, (1, PC), 1)
    mm = (dest_sc[...] == pcols).astype(jnp.float32)
    tw = tokw_sc[...]
    hit = jnp.sum(mm, axis=0)
    tok_c = jnp.sum(mm * tw[:, 0:1], axis=0)
    ws_c = jnp.sum(mm * tw[:, 1:2], axis=0)
    dummy = (pcols[0, :] & (N - 1)).astype(jnp.float32)
    tok_c = tok_c + (1.0 - hit) * dummy
    tok_ref[...] = tok_c.reshape(PC, 1).astype(jnp.int32)
    ws_ref[...] = ws_c.reshape(PC, 1)


def _gate_call(xf, gwt):
    return pl.pallas_call(
        _gate_body,
        grid=(PCH,),
        in_specs=[
            pl.BlockSpec((N, D), lambda g: (0, 0)),
            pl.BlockSpec((D, E), lambda g: (0, 0)),
        ],
        out_specs=[
            pl.BlockSpec((A, 1), lambda g: (0, 0)),
            pl.BlockSpec((PC, 1), lambda g: (g, 0)),
            pl.BlockSpec((PC, 1), lambda g: (g, 0)),
            pl.BlockSpec((NBLK, 1), lambda g: (0, 0)),
            pl.BlockSpec((1, 1), lambda g: (0, 0)),
        ],
        out_shape=[
            jax.ShapeDtypeStruct((A, 1), jnp.int32),
            jax.ShapeDtypeStruct((P, 1), jnp.int32),
            jax.ShapeDtypeStruct((P, 1), jnp.float32),
            jax.ShapeDtypeStruct((NBLK, 1), jnp.int32),
            jax.ShapeDtypeStruct((1, 1), jnp.int32),
        ],
        scratch_shapes=[
            pltpu.VMEM((A, 1), jnp.int32),
            pltpu.VMEM((A, 2), jnp.float32),
        ],
    )(xf, gwt)


def _ffn_body(eog_ref, nblk_ref, xs_ref, w1_ref, w2_ref, ws_ref, out_ref,
              acc_ref, xsb_ref, w1b_ref, w2b_ref):
    k = pl.program_id(0)
    g = pl.program_id(1)
    base = g * BLK

    gprev = jnp.maximum(g - 1, 0)
    wchanged = jnp.logical_or(g == 0, eog_ref[g] != eog_ref[gprev])

    @pl.when(wchanged)
    def _():
        w1b_ref[...] = w1_ref[0].astype(jnp.bfloat16)
        w2b_ref[...] = w2_ref[0].astype(jnp.bfloat16)

    @pl.when(jnp.logical_and(k == 0, g < nblk_ref[0]))
    def _():
        xsb_ref[pl.ds(base, BLK), :] = xs_ref[...].astype(jnp.bfloat16)

    @pl.when(g < nblk_ref[0])
    def _():
        xb = xsb_ref[pl.ds(base, BLK), :]
        h = lax.dot_general(xb, w1b_ref[...], (((1,), (1,)), ((), ())),
                            preferred_element_type=jnp.float32)
        h = 0.5 * h * (1.0 + lax.erf(h * 0.7071067811865476))
        o = lax.dot_general(h.astype(jnp.bfloat16), w2b_ref[...],
                            (((1,), (1,)), ((), ())),
                            preferred_element_type=jnp.float32)

        @pl.when(k == 0)
        def _():
            acc_ref[pl.ds(base, BLK), :] = o.astype(jnp.bfloat16)

        @pl.when(jnp.logical_and(k > 0, k < KCH - 1))
        def _():
            acc_ref[pl.ds(base, BLK), :] = (
                acc_ref[pl.ds(base, BLK), :].astype(jnp.float32) + o
            ).astype(jnp.bfloat16)

        @pl.when(k == KCH - 1)
        def _():
            wsv = ws_ref[0, 0, :].reshape(BLK, 1)
            out_ref[...] = (
                acc_ref[pl.ds(base, BLK), :].astype(jnp.float32) + o) * wsv


def _ffn_call(eog, nblk, xs, W1, W2, ws3):
    grid_spec = pltpu.PrefetchScalarGridSpec(
        num_scalar_prefetch=2,
        grid=(KCH, NBLK),
        in_specs=[
            pl.BlockSpec((BLK, D),
                         lambda k, g, eog, nblk: (jnp.where(k == 0, g, NBLK - 1), 0)),
            pl.BlockSpec((1, DC, D), lambda k, g, eog, nblk: (eog[g], k, 0)),
            pl.BlockSpec((1, D, DC), lambda k, g, eog, nblk: (eog[g], 0, k)),
            pl.BlockSpec((1, 1, BLK), lambda k, g, eog, nblk: (g, 0, 0)),
        ],
        out_specs=pl.BlockSpec(
            (BLK, D), lambda k, g, eog, nblk: (jnp.where(k == KCH - 1, g, 0), 0)),
        scratch_shapes=[
            pltpu.VMEM((P, D), jnp.bfloat16),
            pltpu.VMEM((P, D), jnp.bfloat16),
            pltpu.VMEM((DC, D), jnp.bfloat16),
            pltpu.VMEM((D, DC), jnp.bfloat16),
        ],
    )
    return pl.pallas_call(
        _ffn_body,
        grid_spec=grid_spec,
        out_shape=jax.ShapeDtypeStruct((P, D), jnp.float32),
        compiler_params=pltpu.CompilerParams(
            dimension_semantics=("arbitrary", "arbitrary")),
    )(eog, nblk, xs, W1, W2, ws3)


_SC_NC = 2
_SC_NS = 16
_SC_NW = _SC_NC * _SC_NS


def _sc_gather_rows(table, idx1d, rows, ncols, dtype):
    mesh = plsc.VectorSubcoreMesh(core_axis_name="core",
                                  subcore_axis_name="subcore")
    per_w = rows // _SC_NW
    bytes_per_row = ncols * jnp.dtype(dtype).itemsize
    max_chunk = (160 * 1024) // bytes_per_row
    chunk = max(c for c in (8, 16, 24, 32, 40, 64, 80)
                if c <= max_chunk and per_w % c == 0)
    nchunk = per_w // chunk

    @functools.partial(
        pl.kernel,
        out_type=jax.ShapeDtypeStruct((rows, ncols), dtype),
        mesh=mesh,
        scratch_types=[
            pltpu.VMEM((per_w,), jnp.int32),
            pltpu.VMEM((chunk, ncols), dtype),
            pltpu.VMEM((chunk, ncols), dtype),
            pltpu.SemaphoreType.DMA,
            pltpu.SemaphoreType.DMA,
        ])
    def kern(x_hbm, i_hbm, o_hbm, idx_v, rv0, rv1, sem0, sem1):
        wid = lax.axis_index("subcore") * _SC_NC + lax.axis_index("core")
        base = wid * per_w
        pltpu.sync_copy(i_hbm.at[pl.ds(base, per_w)], idx_v)
        bufs = (rv0, rv1)
        sems = (sem0, sem1)

        def start(c):
            b = c % 2
            return pltpu.async_copy(
                x_hbm.at[idx_v.at[pl.ds(c * chunk, chunk)]], bufs[b], sems[b])

        handles = [start(0)]
        for c in range(nchunk):
            if c + 1 < nchunk:
                handles.append(start(c + 1))
            handles[c].wait()
            pltpu.sync_copy(bufs[c % 2],
                            o_hbm.at[pl.ds(base + c * chunk, chunk)])

    return kern(table, idx1d)


def _add_body(a_ref, b_ref, o_ref):
    o_ref[...] = a_ref[...] + b_ref[...]


_ADD_BLK = 512


def _add_call(r):
    return pl.pallas_call(
        _add_body,
        grid=(N // _ADD_BLK,),
        in_specs=[
            pl.BlockSpec((_ADD_BLK, D), lambda g: (g, 0)),
            pl.BlockSpec((_ADD_BLK, D), lambda g: (g + N // _ADD_BLK, 0)),
        ],
        out_specs=pl.BlockSpec((_ADD_BLK, D), lambda g: (g, 0)),
        out_shape=jax.ShapeDtypeStruct((N, D), jnp.float32),
    )(r, r)


def kernel(x, gate_w, W1, W2):
    b, t, h, w, d = x.shape
    xf = x.reshape(N, D)
    gwt = gate_w.T

    dest, tok, ws, eog, nblk = _gate_call(xf, gwt)
    dest = dest.reshape(A)
    eog = eog.reshape(NBLK)
    nblk = nblk.reshape(1)

    xs = _sc_gather_rows(xf, tok.reshape(P), P, D, jnp.float32)
    ws3 = ws.reshape(NBLK, 1, BLK)
    out_s = _ffn_call(eog, nblk, xs, W1, W2, ws3)

    r = _sc_gather_rows(out_s, dest, A, D, jnp.float32)
    y = _add_call(r)
    return y.reshape(b, t, h, w, d)


# speedup vs baseline: 3.0119x; 1.0046x over previous
import functools

import jax
import jax.numpy as jnp
from jax import lax
from jax.experimental import pallas as pl
from jax.experimental.pallas import tpu as pltpu
from jax.experimental.pallas import tpu_sc as plsc

E = 8
TOPK = 2
N = 2048
D = 1024
DFF = 4096
A = N * TOPK
BLK = 256
NBLK = (A + E * BLK) // BLK
P = NBLK * BLK
DC = 1024
KCH = DFF // DC
PC = 1024
PCH = P // PC


def _exclusive_ladder_cumsum(x, axis, length):
    sh = 1
    while sh < length:
        if axis == 0:
            pad = jnp.zeros((sh,) + x.shape[1:], x.dtype)
            x = x + jnp.concatenate([pad, x[:-sh]], axis=0)
        else:
            pad = jnp.zeros(x.shape[:1] + (sh,), x.dtype)
            x = x + jnp.concatenate([pad, x[:, :-sh]], axis=1)
        sh *= 2
    return x


def _gate_body(xf_ref, gwt_ref, dest_ref, tok_ref, ws_ref, eog_ref, nblk_ref,
               dest_sc, tokw_sc):
    g = pl.program_id(0)

    @pl.when(g == 0)
    def _():
        xf = xf_ref[...]
        logits = jnp.dot(xf, gwt_ref[...],
                         preferred_element_type=jnp.float32)
        m = jnp.max(logits, axis=1, keepdims=True)
        p = jnp.exp(logits - m)
        probs = p / jnp.sum(p, axis=1, keepdims=True)

        col = lax.broadcasted_iota(jnp.int32, (N, E), 1)
        m1 = jnp.max(probs, axis=1, keepdims=True)
        a1 = jnp.min(jnp.where(probs == m1, col, E), axis=1, keepdims=True)
        probs2 = jnp.where(col == a1, -1.0, probs)
        m2 = jnp.max(probs2, axis=1, keepdims=True)
        a2 = jnp.min(jnp.where(probs2 == m2, col, E), axis=1, keepdims=True)

        ef = jnp.concatenate([a1, a2], axis=0)
        wf = jnp.concatenate([m1, m2], axis=0)
        oh = (ef == lax.broadcasted_iota(jnp.int32, (A, E), 1)
              ).astype(jnp.float32)
        csum = _exclusive_ladder_cumsum(oh, 0, A)
        rank = jnp.sum(oh * csum, axis=1, keepdims=True) - 1.0
        cnt = csum[A - 1:A, :]
        blocks = jnp.floor((cnt + (BLK - 1)) * (1.0 / BLK))
        ic = _exclusive_ladder_cumsum(blocks, 1, E)
        po = BLK * (ic - blocks)
        dest = rank + jnp.sum(oh * po, axis=1, keepdims=True)
        tb = jnp.sum(blocks)
        cole = lax.broadcasted_iota(jnp.int32, (1, E), 1).astype(jnp.float32)
        le = jnp.max(jnp.where(cnt > 0.0, cole, -1.0))

        grow = lax.broadcasted_iota(jnp.int32, (NBLK, 1), 0).astype(jnp.float32)
        eog_raw = jnp.sum((BLK * ic <= BLK * grow).astype(jnp.float32),
                          axis=1, keepdims=True)
        eog = jnp.where(grow < tb, eog_raw, le)
        eog = jnp.clip(eog, 0.0, float(E - 1))

        irow = lax.broadcasted_iota(jnp.int32, (A, 1), 0)
        tokf = (irow & (N - 1)).astype(jnp.float32)

        dest_sc[...] = dest.astype(jnp.int32)
        tokw_sc[...] = jnp.concatenate([tokf, wf], axis=1)
        dest_ref[...] = dest.astype(jnp.int32)
        eog_ref[...] = eog.astype(jnp.int32)
        nblk_ref[...] = tb.astype(jnp.int32).reshape(1, 1)

    pcols = PC * g + lax.broadcasted_iota(jnp.int32, (1, PC), 1)
    mm = (dest_sc[...] == pcols).astype(jnp.float32)
    tw = tokw_sc[...]
    hit = jnp.sum(mm, axis=0)
    tok_c = jnp.sum(mm * tw[:, 0:1], axis=0)
    ws_c = jnp.sum(mm * tw[:, 1:2], axis=0)
    dummy = (pcols[0, :] & (N - 1)).astype(jnp.float32)
    tok_c = tok_c + (1.0 - hit) * dummy
    tok_ref[...] = tok_c.reshape(PC, 1).astype(jnp.int32)
    ws_ref[...] = ws_c.reshape(PC, 1)


def _gate_call(xf, gwt):
    return pl.pallas_call(
        _gate_body,
        grid=(PCH,),
        in_specs=[
            pl.BlockSpec((N, D), lambda g: (0, 0)),
            pl.BlockSpec((D, E), lambda g: (0, 0)),
        ],
        out_specs=[
            pl.BlockSpec((A, 1), lambda g: (0, 0)),
            pl.BlockSpec((PC, 1), lambda g: (g, 0)),
            pl.BlockSpec((PC, 1), lambda g: (g, 0)),
            pl.BlockSpec((NBLK, 1), lambda g: (0, 0)),
            pl.BlockSpec((1, 1), lambda g: (0, 0)),
        ],
        out_shape=[
            jax.ShapeDtypeStruct((A, 1), jnp.int32),
            jax.ShapeDtypeStruct((P, 1), jnp.int32),
            jax.ShapeDtypeStruct((P, 1), jnp.float32),
            jax.ShapeDtypeStruct((NBLK, 1), jnp.int32),
            jax.ShapeDtypeStruct((1, 1), jnp.int32),
        ],
        scratch_shapes=[
            pltpu.VMEM((A, 1), jnp.int32),
            pltpu.VMEM((A, 2), jnp.float32),
        ],
    )(xf, gwt)


def _ffn_body(eog_ref, nblk_ref, xs_ref, w1_ref, w2_ref, ws_ref, out_ref,
              acc_ref, xsb_ref, w1b_ref, w2b_ref):
    k = pl.program_id(0)
    g = pl.program_id(1)
    base = g * BLK

    gprev = jnp.maximum(g - 1, 0)
    wchanged = jnp.logical_or(g == 0, eog_ref[g] != eog_ref[gprev])

    @pl.when(wchanged)
    def _():
        w1b_ref[...] = w1_ref[0].astype(jnp.bfloat16)
        w2b_ref[...] = w2_ref[0].astype(jnp.bfloat16)

    @pl.when(jnp.logical_and(k == 0, g < nblk_ref[0]))
    def _():
        xsb_ref[pl.ds(base, BLK), :] = xs_ref[...].astype(jnp.bfloat16)

    @pl.when(g < nblk_ref[0])
    def _():
        xb = xsb_ref[pl.ds(base, BLK), :]
        h = lax.dot_general(xb, w1b_ref[...], (((1,), (1,)), ((), ())),
                            preferred_element_type=jnp.float32)
        h = 0.5 * h * (1.0 + lax.erf(h * 0.7071067811865476))
        o = lax.dot_general(h.astype(jnp.bfloat16), w2b_ref[...],
                            (((1,), (1,)), ((), ())),
                            preferred_element_type=jnp.float32)

        @pl.when(k == 0)
        def _():
            acc_ref[pl.ds(base, BLK), :] = o.astype(jnp.bfloat16)

        @pl.when(jnp.logical_and(k > 0, k < KCH - 1))
        def _():
            acc_ref[pl.ds(base, BLK), :] = (
                acc_ref[pl.ds(base, BLK), :].astype(jnp.float32) + o
            ).astype(jnp.bfloat16)

        @pl.when(k == KCH - 1)
        def _():
            wsv = ws_ref[0, 0, :].reshape(BLK, 1)
            out_ref[...] = (
                acc_ref[pl.ds(base, BLK), :].astype(jnp.float32) + o) * wsv


def _ffn_call(eog, nblk, xs, W1, W2, ws3):
    grid_spec = pltpu.PrefetchScalarGridSpec(
        num_scalar_prefetch=2,
        grid=(KCH, NBLK),
        in_specs=[
            pl.BlockSpec((BLK, D),
                         lambda k, g, eog, nblk: (jnp.where(k == 0, g, NBLK - 1), 0)),
            pl.BlockSpec((1, DC, D), lambda k, g, eog, nblk: (eog[g], k, 0)),
            pl.BlockSpec((1, D, DC), lambda k, g, eog, nblk: (eog[g], 0, k)),
            pl.BlockSpec((1, 1, BLK), lambda k, g, eog, nblk: (g, 0, 0)),
        ],
        out_specs=pl.BlockSpec(
            (BLK, D), lambda k, g, eog, nblk: (jnp.where(k == KCH - 1, g, 0), 0)),
        scratch_shapes=[
            pltpu.VMEM((P, D), jnp.bfloat16),
            pltpu.VMEM((P, D), jnp.bfloat16),
            pltpu.VMEM((DC, D), jnp.bfloat16),
            pltpu.VMEM((D, DC), jnp.bfloat16),
        ],
    )
    return pl.pallas_call(
        _ffn_body,
        grid_spec=grid_spec,
        out_shape=jax.ShapeDtypeStruct((P, D), jnp.float32),
        compiler_params=pltpu.CompilerParams(
            dimension_semantics=("arbitrary", "arbitrary")),
    )(eog, nblk, xs, W1, W2, ws3)


_SC_NC = 2
_SC_NS = 16
_SC_NW = _SC_NC * _SC_NS


def _sc_gather_rows(table, idx1d, rows, ncols, dtype):
    mesh = plsc.VectorSubcoreMesh(core_axis_name="core",
                                  subcore_axis_name="subcore")
    per_w = rows // _SC_NW
    bytes_per_row = ncols * jnp.dtype(dtype).itemsize
    max_chunk = (160 * 1024) // bytes_per_row
    chunk = max(c for c in (8, 16, 24, 32, 40, 64, 80)
                if c <= max_chunk and per_w % c == 0)
    nchunk = per_w // chunk

    @functools.partial(
        pl.kernel,
        out_type=jax.ShapeDtypeStruct((rows, ncols), dtype),
        mesh=mesh,
        scratch_types=[
            pltpu.VMEM((per_w,), jnp.int32),
            pltpu.VMEM((chunk, ncols), dtype),
            pltpu.VMEM((chunk, ncols), dtype),
            pltpu.SemaphoreType.DMA,
            pltpu.SemaphoreType.DMA,
        ])
    def kern(x_hbm, i_hbm, o_hbm, idx_v, rv0, rv1, sem0, sem1):
        wid = lax.axis_index("subcore") * _SC_NC + lax.axis_index("core")
        base = wid * per_w
        pltpu.sync_copy(i_hbm.at[pl.ds(base, per_w)], idx_v)
        bufs = (rv0, rv1)
        sems = (sem0, sem1)

        def start(c):
            b = c % 2
            return pltpu.async_copy(
                x_hbm.at[idx_v.at[pl.ds(c * chunk, chunk)]], bufs[b], sems[b])

        handles = [start(0)]
        for c in range(nchunk):
            if c + 1 < nchunk:
                handles.append(start(c + 1))
            handles[c].wait()
            pltpu.sync_copy(bufs[c % 2],
                            o_hbm.at[pl.ds(base + c * chunk, chunk)])

    return kern(table, idx1d)


def _add_body(a_ref, b_ref, o_ref):
    o_ref[...] = a_ref[...] + b_ref[...]


_ADD_BLK = 512


def _add_call(r):
    return pl.pallas_call(
        _add_body,
        grid=(N // _ADD_BLK,),
        in_specs=[
            pl.BlockSpec((_ADD_BLK, D), lambda g: (g, 0)),
            pl.BlockSpec((_ADD_BLK, D), lambda g: (g + N // _ADD_BLK, 0)),
        ],
        out_specs=pl.BlockSpec((_ADD_BLK, D), lambda g: (g, 0)),
        out_shape=jax.ShapeDtypeStruct((N, D), jnp.float32),
    )(r, r)


def kernel(x, gate_w, W1, W2):
    b, t, h, w, d = x.shape
    xf = x.reshape(N, D)
    gwt = gate_w.T

    dest, tok, ws, eog, nblk = _gate_call(xf, gwt)
    dest = dest.reshape(A)
    eog = eog.reshape(NBLK)
    nblk = nblk.reshape(1)

    xs = _sc_gather_rows(xf, tok.reshape(P), P, D, jnp.float32)
    ws3 = ws.reshape(NBLK, 1, BLK)
    out_s = _ffn_call(eog, nblk, xs, W1, W2, ws3)

    r = _sc_gather_rows(out_s, dest, A, D, jnp.float32)
    y = _add_call(r)
    return y.reshape(b, t, h, w, d)
